# baseline (speedup 1.0000x reference)
import jax
import jax.numpy as jnp
from jax.experimental import pallas as pl
from jax.experimental.pallas import tpu as pltpu

_EP = 128


def _moe_body(r_ref, x_ref, wr_ref, br_ref, we_ref, be_ref, out_ref,
              acc_ref, un_ref):
    e = pl.program_id(1)
    ne = pl.num_programs(1)

    @pl.when(e == 0)
    def _():
        logits = jnp.dot(r_ref[...], wr_ref[...],
                         preferred_element_type=jnp.float32)
        logits = logits + br_ref[...]
        m = jnp.max(logits, axis=-1, keepdims=True)
        u = jnp.exp(logits - m)
        un = u / jnp.sum(u, axis=-1, keepdims=True)
        un_ref[...] = un
        acc_ref[...] = jnp.dot(un, be_ref[...],
                               preferred_element_type=jnp.float32)

    un = un_ref[...]
    lane = jax.lax.broadcasted_iota(jnp.int32, un.shape, 1)
    w_e = jnp.sum(jnp.where(lane == e, un, 0.0), axis=1, keepdims=True)
    y = jnp.dot(x_ref[...], we_ref[0], preferred_element_type=jnp.float32)
    new_acc = acc_ref[...] + w_e * y

    @pl.when(e != ne - 1)
    def _():
        acc_ref[...] = new_acc

    @pl.when(e == ne - 1)
    def _():
        out_ref[...] = new_acc


def kernel(router_input, x, Wr, br, We, be):
    n, d = x.shape
    n_exp = We.shape[0]
    bn = 512

    rb = router_input.astype(jnp.bfloat16)
    xb = x.astype(jnp.bfloat16)
    web = We.astype(jnp.bfloat16)
    wrp = jnp.zeros((d, _EP), jnp.bfloat16).at[:, :n_exp].set(
        Wr.astype(jnp.bfloat16))
    brp = jnp.full((1, _EP), -jnp.inf, jnp.float32).at[0, :n_exp].set(br)
    bep = jnp.zeros((_EP, d), jnp.float32).at[:n_exp].set(be)

    return pl.pallas_call(
        _moe_body,
        grid=(n // bn, n_exp),
        in_specs=[
            pl.BlockSpec((bn, d), lambda i, e: (i, 0)),
            pl.BlockSpec((bn, d), lambda i, e: (i, 0)),
            pl.BlockSpec((d, _EP), lambda i, e: (0, 0)),
            pl.BlockSpec((1, _EP), lambda i, e: (0, 0)),
            pl.BlockSpec((1, d, d), lambda i, e: (e, 0, 0)),
            pl.BlockSpec((_EP, d), lambda i, e: (0, 0)),
        ],
        out_specs=pl.BlockSpec((bn, d), lambda i, e: (i, 0)),
        out_shape=jax.ShapeDtypeStruct((n, d), jnp.float32),
        scratch_shapes=[
            pltpu.VMEM((bn, d), jnp.float32),
            pltpu.VMEM((bn, _EP), jnp.float32),
        ],
        compiler_params=pltpu.CompilerParams(
            dimension_semantics=("parallel", "arbitrary"),
        ),
    )(rb, xb, wrp, brp, web, bep)


# speedup vs baseline: 1.2554x; 1.2554x over previous
import jax
import jax.numpy as jnp
from jax.experimental import pallas as pl
from jax.experimental.pallas import tpu as pltpu

_EP = 128


def _moe_body(r_ref, x_ref, wr_ref, br_ref, we_ref, be_ref, out_ref):
    n_exp = we_ref.shape[0]
    logits = jnp.dot(r_ref[...], wr_ref[...], preferred_element_type=jnp.float32)
    logits = logits + br_ref[...]
    m = jnp.max(logits, axis=-1, keepdims=True)
    u = jnp.exp(logits - m)
    denom = jnp.sum(u, axis=-1, keepdims=True)

    x = x_ref[...]
    acc = jnp.dot(u.astype(jnp.bfloat16), be_ref[...],
                  preferred_element_type=jnp.float32)
    for e in range(n_exp):
        y = jnp.dot(x, we_ref[e], preferred_element_type=jnp.float32)
        acc = acc + u[:, e : e + 1] * y
    out_ref[...] = acc / denom


def kernel(router_input, x, Wr, br, We, be):
    n, d = x.shape
    n_exp = We.shape[0]
    bn = 512

    rb = router_input.astype(jnp.bfloat16)
    xb = x.astype(jnp.bfloat16)
    web = We.astype(jnp.bfloat16)
    wrp = jnp.zeros((d, _EP), jnp.bfloat16).at[:, :n_exp].set(
        Wr.astype(jnp.bfloat16))
    brp = jnp.full((1, _EP), -jnp.inf, jnp.float32).at[0, :n_exp].set(br)
    bep = jnp.zeros((_EP, d), jnp.bfloat16).at[:n_exp].set(
        be.astype(jnp.bfloat16))

    return pl.pallas_call(
        _moe_body,
        grid=(n // bn,),
        in_specs=[
            pl.BlockSpec((bn, d), lambda i: (i, 0)),
            pl.BlockSpec((bn, d), lambda i: (i, 0)),
            pl.BlockSpec((d, _EP), lambda i: (0, 0)),
            pl.BlockSpec((1, _EP), lambda i: (0, 0)),
            pl.BlockSpec((n_exp, d, d), lambda i: (0, 0, 0)),
            pl.BlockSpec((_EP, d), lambda i: (0, 0)),
        ],
        out_specs=pl.BlockSpec((bn, d), lambda i: (i, 0)),
        out_shape=jax.ShapeDtypeStruct((n, d), jnp.float32),
        compiler_params=pltpu.CompilerParams(
            dimension_semantics=("arbitrary",),
        ),
    )(rb, xb, wrp, brp, web, bep)


# speedup vs baseline: 1.6125x; 1.2845x over previous
import jax
import jax.numpy as jnp
from jax.experimental import pallas as pl
from jax.experimental.pallas import tpu as pltpu

_EP = 128


def _moe_body(r_ref, x_ref, wr_ref, br_ref, we_ref, be_ref, out_ref):
    n_exp = we_ref.shape[0]
    rb = r_ref[...].astype(jnp.bfloat16)
    logits = jnp.dot(rb, wr_ref[...], preferred_element_type=jnp.float32)
    logits = logits + br_ref[...]
    m = jnp.max(logits, axis=-1, keepdims=True)
    u = jnp.exp(logits - m)
    denom = jnp.sum(u, axis=-1, keepdims=True)

    x = x_ref[...].astype(jnp.bfloat16)
    acc = jnp.dot(u.astype(jnp.bfloat16), be_ref[...],
                  preferred_element_type=jnp.float32)
    for e in range(n_exp):
        w = we_ref[e].astype(jnp.bfloat16)
        acc = acc + u[:, e : e + 1] * jnp.dot(
            x, w, preferred_element_type=jnp.float32)
    out_ref[...] = acc / denom


def kernel(router_input, x, Wr, br, We, be):
    n, d = x.shape
    n_exp = We.shape[0]
    bn = 512

    wrp = jnp.zeros((d, _EP), jnp.bfloat16).at[:, :n_exp].set(
        Wr.astype(jnp.bfloat16))
    brp = jnp.full((1, _EP), -jnp.inf, jnp.float32).at[0, :n_exp].set(br)
    bep = jnp.zeros((_EP, d), jnp.bfloat16).at[:n_exp].set(
        be.astype(jnp.bfloat16))

    return pl.pallas_call(
        _moe_body,
        grid=(n // bn,),
        in_specs=[
            pl.BlockSpec((bn, d), lambda i: (i, 0)),
            pl.BlockSpec((bn, d), lambda i: (i, 0)),
            pl.BlockSpec((d, _EP), lambda i: (0, 0)),
            pl.BlockSpec((1, _EP), lambda i: (0, 0)),
            pl.BlockSpec((n_exp, d, d), lambda i: (0, 0, 0)),
            pl.BlockSpec((_EP, d), lambda i: (0, 0)),
        ],
        out_specs=pl.BlockSpec((bn, d), lambda i: (i, 0)),
        out_shape=jax.ShapeDtypeStruct((n, d), jnp.float32),
        compiler_params=pltpu.CompilerParams(
            dimension_semantics=("arbitrary",),
        ),
    )(router_input, x, wrp, brp, We, bep)
